# Initial kernel scaffold; baseline (speedup 1.0000x reference)
import jax
import jax.numpy as jnp
from jax.experimental import pallas as pl


def kernel(x, mask, table, W1, b1, W2, b2):
    raise NotImplementedError("write your pallas kernel here")


# speedup vs baseline: 1.0522x; 1.0522x over previous
import functools

import jax
import jax.numpy as jnp
from jax import lax
from jax.experimental import pallas as pl
from jax.experimental.pallas import tpu as pltpu
from jax.experimental.pallas import tpu_sc as plsc

NUM_WORKERS = 32
CHUNK = 40


def _make_sc_pool(batch, seq, vocab, embed):
    assert batch % NUM_WORKERS == 0
    b_per_w = batch // NUM_WORKERS
    assert seq % CHUNK == 0
    chunks_per_row = seq // CHUNK
    chunks_per_w = b_per_w * chunks_per_row
    assert embed % 16 == 0
    nvec = embed // 16

    mesh = plsc.VectorSubcoreMesh(core_axis_name="c", subcore_axis_name="s",
                                  num_cores=2, num_subcores=16)

    @functools.partial(
        pl.kernel,
        out_type=jax.ShapeDtypeStruct((batch, embed), jnp.float32),
        mesh=mesh,
        scratch_types=[
            pltpu.VMEM((chunks_per_w, CHUNK), jnp.int32),
            pltpu.VMEM((seq, embed), jnp.float32),
            pltpu.VMEM((seq, embed), jnp.float32),
            pltpu.VMEM((b_per_w, embed), jnp.float32),
            pltpu.SemaphoreType.DMA,
            pltpu.SemaphoreType.DMA,
        ],
        compiler_params=pltpu.CompilerParams(use_tc_tiling_on_sc=False),
    )
    def sc_pool(x_hbm, table_hbm, out_hbm, idx_v, buf_a, buf_b, sout_v,
                sem_a, sem_b):
        wid = lax.axis_index("s") * 2 + lax.axis_index("c")
        base = wid * b_per_w
        bufs = (buf_a, buf_b)
        sems = (sem_a, sem_b)

        pltpu.sync_copy(x_hbm.at[wid], idx_v)

        def fire(row, buf, sem):
            cbase = row * chunks_per_row
            for c in range(chunks_per_row):
                pltpu.async_copy(
                    table_hbm.at[idx_v.at[cbase + c]],
                    buf.at[pl.ds(c * CHUNK, CHUNK)],
                    sem,
                )

        def drain(buf, sem):
            pltpu.make_async_copy(table_hbm.at[pl.ds(0, seq)], buf, sem).wait()

        def accumulate(row, buf):
            def step(t, accs):
                rbase = t * 8
                out = []
                for k in range(nvec):
                    sl = pl.ds(k * 16, 16)
                    l = [buf[rbase + r, sl] for r in range(8)]
                    s = ((l[0] + l[1]) + (l[2] + l[3])) + \
                        ((l[4] + l[5]) + (l[6] + l[7]))
                    out.append(accs[k] + s)
                return tuple(out)

            zeros = tuple(jnp.zeros((16,), jnp.float32) for _ in range(nvec))
            accs = lax.fori_loop(0, seq // 8, step, zeros)
            for k in range(nvec):
                sout_v[row, pl.ds(k * 16, 16)] = accs[k]

        fire(0, bufs[0], sems[0])

        @pl.loop(0, b_per_w, step=2)
        def _row_loop(i):
            for b in range(2):
                row = i + b
                nxt = row + 1

                @pl.when(nxt < b_per_w)
                def _():
                    fire(nxt, bufs[1 - b], sems[1 - b])

                drain(bufs[b], sems[b])
                accumulate(row, bufs[b])

        pltpu.sync_copy(sout_v, out_hbm.at[pl.ds(base, b_per_w)])

    return sc_pool


def _mlp_body(ssum_ref, mask_ref, w1_ref, b1_ref, w2_ref, b2_ref, out_ref):
    denom = jnp.maximum(jnp.sum(mask_ref[...], axis=1, keepdims=True), 1.0)
    s = ssum_ref[...] / denom
    h = jnp.dot(s, w1_ref[...], preferred_element_type=jnp.float32)
    h = jnp.maximum(h + b1_ref[...], 0.0)
    out_ref[...] = jnp.dot(h, w2_ref[...],
                           preferred_element_type=jnp.float32) + b2_ref[...]


def kernel(x, mask, table, W1, b1, W2, b2):
    batch, seq = x.shape
    vocab, embed = table.shape
    hidden = W1.shape[1]
    nout = W2.shape[1]

    xr = x.astype(jnp.int32).reshape(NUM_WORKERS, (batch // NUM_WORKERS) * (seq // CHUNK), CHUNK)
    ssum = _make_sc_pool(batch, seq, vocab, embed)(xr, table)

    blk = 512
    grid = (batch // blk,)
    out = pl.pallas_call(
        _mlp_body,
        grid=grid,
        in_specs=[
            pl.BlockSpec((blk, embed), lambda i: (i, 0)),
            pl.BlockSpec((blk, seq), lambda i: (i, 0)),
            pl.BlockSpec((embed, hidden), lambda i: (0, 0)),
            pl.BlockSpec((1, hidden), lambda i: (0, 0)),
            pl.BlockSpec((hidden, nout), lambda i: (0, 0)),
            pl.BlockSpec((1, nout), lambda i: (0, 0)),
        ],
        out_specs=pl.BlockSpec((blk, nout), lambda i: (i, 0)),
        out_shape=jax.ShapeDtypeStruct((batch, nout), jnp.float32),
    )(ssum, mask, W1, b1.reshape(1, -1), W2, b2.reshape(1, -1))
    return out
